# baseline (speedup 1.0000x reference)
import math
import functools

import jax
import jax.numpy as jnp
from jax import lax
from jax.experimental import pallas as pl
from jax.experimental.pallas import tpu as pltpu
from jax.experimental.pallas import tpu_sc as plsc

B = 1024
M = 200
G = 8
K = 10
LAM = 0.5
NUM_MC = 25

NW = 32
RW = B // NW
PAIRS = RW * NUM_MC
NGRP = PAIRS // 16
LOG_KFACT = math.log(float(math.factorial(K)))
LN2 = 0.6931471805599453


def _log16(x):
    b = lax.bitcast_convert_type(x, jnp.int32)
    e = (b >> 23) - 127
    m = lax.bitcast_convert_type((b & 0x007FFFFF) | 0x3F800000, jnp.float32)
    big = m > 1.4142135
    m = jnp.where(big, m * 0.5, m)
    e = jnp.where(big, e + 1, e)
    t = m - 1.0
    s = t / (t + 2.0)
    s2 = s * s
    p = 2.0 * s * (1.0 + s2 * (1.0 / 3.0 + s2 * (1.0 / 5.0 + s2 * (1.0 / 7.0 + s2 * (1.0 / 9.0)))))
    return e.astype(jnp.float32) * LN2 + p


def _sc_body(scores_h, rel_h, eth_h, g_h, sig_h, out_h,
             scores_v, rel_v, eth_v, probs_v, g_v, sig_v, zt_v, qbuf_v,
             tp_v, srel_v, outbuf_v):
    wid = lax.axis_index("s") * 2 + lax.axis_index("c")
    lane = lax.iota(jnp.int32, 16)
    zeros_f = jnp.zeros((16,), jnp.float32)
    zeros_i = jnp.zeros((16,), jnp.int32)
    neg_inf = jnp.full((16,), -jnp.inf, jnp.float32)

    pltpu.sync_copy(scores_h.at[pl.ds(wid * RW * M, RW * M)], scores_v)
    pltpu.sync_copy(rel_h.at[pl.ds(wid * RW * M, RW * M)], rel_v)
    pltpu.sync_copy(eth_h.at[pl.ds(wid * RW * M * G, RW * M * G)], eth_v)

    for bat in range(2):
        rbase = (bat * 16 + lane) * M

        def mx_body(j, mv):
            jv = jnp.full((16,), j, jnp.int32)
            return jnp.maximum(mv, plsc.load_gather(scores_v, [rbase + jv]))
        mv = lax.fori_loop(0, M, mx_body, neg_inf)

        def ex_body(j, carry):
            te, sr = carry
            jv = jnp.full((16,), j, jnp.int32)
            v = plsc.load_gather(scores_v, [rbase + jv])
            ev = jnp.exp(v - mv)
            plsc.store_scatter(probs_v, [rbase + jv], ev)
            rv = plsc.load_gather(rel_v, [rbase + jv])
            return (te + ev, sr + rv)
        te, sr = lax.fori_loop(0, M, ex_body, (zeros_f, zeros_f))

        inv_te = 1.0 / te

        def pscale(j, tp):
            jv = jnp.full((16,), j, jnp.int32)
            pv = plsc.load_gather(probs_v, [rbase + jv]) * inv_te
            plsc.store_scatter(probs_v, [rbase + jv], pv)
            return tp + pv
        tp = lax.fori_loop(0, M, pscale, zeros_f)

        tp_v[pl.ds(bat * 16, 16)] = tp
        srel_v[pl.ds(bat * 16, 16)] = sr

    lbase = lane * M

    def grp_body(grp, acc):
        pair0 = wid * PAIRS + grp * 16
        pltpu.sync_copy(g_h.at[pl.ds(pair0 * M, 16 * M)], g_v)
        pltpu.sync_copy(sig_h.at[pl.ds(pair0 * K, 16 * K)], sig_v)
        p_loc = grp * 16 + lane
        bl = p_loc // NUM_MC
        blM = bl * M

        def zb(j, jv):
            gv = plsc.load_gather(g_v, [lbase + jv])
            sv = plsc.load_gather(scores_v, [blM + jv])
            zt_v[pl.ds(pl.multiple_of(j * 16, 16), 16)] = gv + sv
            return jv + 1
        lax.fori_loop(0, M, zb, zeros_i)

        best_idx = []
        q_list = []
        r_list = []
        for i in range(K):
            def tk(j, carry):
                bv, bi, jv = carry
                v = zt_v[pl.ds(pl.multiple_of(j * 16, 16), 16)]
                gt = v > bv
                return (jnp.where(gt, v, bv), jnp.where(gt, jv, bi), jv + 1)
            bv, bi, _ = lax.fori_loop(0, M, tk, (neg_inf, zeros_i, zeros_i))
            if i < K - 1:
                plsc.store_scatter(zt_v, [bi * 16 + lane], neg_inf)
            best_idx.append(bi)
            qi = plsc.load_gather(probs_v, [blM + bi])
            qbuf_v[pl.ds(i * 16, 16)] = qi
            q_list.append(qi)
            r_list.append(plsc.load_gather(rel_v, [blM + bi]))

        tp_l = plsc.load_gather(tp_v, [bl])
        srel_l = plsc.load_gather(srel_v, [bl])
        inv_srel = jnp.where(srel_l > 0, 1.0 / srel_l, 0.0)
        T_r = srel_l * inv_srel

        sr_top = r_list[0]
        for i in range(1, K):
            sr_top = sr_top + r_list[i]
        delta = 2.0 * (sr_top * inv_srel) - T_r

        pos = sr_top > 0
        inv_sr = jnp.where(pos, 1.0 / sr_top, 0.0)
        w = [jnp.where(pos, r_list[i] * inv_sr, 1.0 / K) for i in range(K)]

        f_list = []
        for gd in range(G):
            fg = zeros_f
            for i in range(K):
                fg = fg + w[i] * plsc.load_gather(
                    eth_v, [(blM + best_idx[i]) * G + gd])
            f_list.append(fg)
        SF = f_list[0]
        for gd in range(1, G):
            SF = SF + f_list[gd]
        inv_SF = 1.0 / SF
        H = zeros_f
        for gd in range(G):
            pj = f_list[gd] * inv_SF
            H = H - jnp.where(pj > 0, pj * _log16(pj), 0.0)

        logq = zeros_f
        logD = zeros_f
        accq = zeros_f
        for i in range(K):
            sig_i = plsc.load_gather(sig_v, [lane * K + i])
            qp = plsc.load_gather(qbuf_v, [sig_i * 16 + lane])
            logq = logq + _log16(qp)
            logD = logD + _log16(tp_l - accq)
            accq = accq + qp
        logprob = LOG_KFACT + logq - logD

        reward = delta + LAM * H
        return acc + logprob * reward

    acc = lax.fori_loop(0, NGRP, grp_body, zeros_f)
    outbuf_v[...] = acc
    pltpu.sync_copy(outbuf_v, out_h.at[wid])


def kernel(scores, relevance, eth_list):
    key = jax.random.key(1234)
    k_sample, k_perm = jax.random.split(key)
    g = jax.random.gumbel(k_sample, (B, NUM_MC, M), dtype=jnp.float32)
    sigma = jnp.argsort(jax.random.uniform(k_perm, (B, NUM_MC, K)), axis=-1)

    mesh = plsc.VectorSubcoreMesh(core_axis_name="c", subcore_axis_name="s")
    run = functools.partial(
        pl.kernel,
        mesh=mesh,
        compiler_params=pltpu.CompilerParams(needs_layout_passes=False),
        out_type=jax.ShapeDtypeStruct((NW, 16), jnp.float32),
        scratch_types=[
            pltpu.VMEM((RW * M,), jnp.float32),
            pltpu.VMEM((RW * M,), jnp.float32),
            pltpu.VMEM((RW * M * G,), jnp.float32),
            pltpu.VMEM((RW * M,), jnp.float32),
            pltpu.VMEM((16 * M,), jnp.float32),
            pltpu.VMEM((16 * K,), jnp.int32),
            pltpu.VMEM((M * 16,), jnp.float32),
            pltpu.VMEM((K * 16,), jnp.float32),
            pltpu.VMEM((RW,), jnp.float32),
            pltpu.VMEM((RW,), jnp.float32),
            pltpu.VMEM((16,), jnp.float32),
        ],
    )(_sc_body)
    part = run(
        scores.reshape(B * M),
        relevance.reshape(B * M),
        eth_list.reshape(B * M * G),
        g.reshape(B * NUM_MC * M),
        sigma.reshape(B * NUM_MC * K).astype(jnp.int32),
    )
    return -jnp.sum(part) / (NUM_MC * B)


# speedup vs baseline: 2.4065x; 2.4065x over previous
import math
import functools

import jax
import jax.numpy as jnp
from jax import lax
from jax.experimental import pallas as pl
from jax.experimental.pallas import tpu as pltpu
from jax.experimental.pallas import tpu_sc as plsc

B = 1024
M = 200
G = 8
K = 10
LAM = 0.5
NUM_MC = 25

NW = 32
RW = B // NW
PAIRS = RW * NUM_MC
NGRP = PAIRS // 16
CH = 20
NCH = M // CH
LOG_KFACT = math.log(float(math.factorial(K)))
LN2 = 0.6931471805599453


def _log16(x):
    b = lax.bitcast_convert_type(x, jnp.int32)
    e = (b >> 23) - 127
    m = lax.bitcast_convert_type((b & 0x007FFFFF) | 0x3F800000, jnp.float32)
    big = m > 1.4142135
    m = jnp.where(big, m * 0.5, m)
    e = jnp.where(big, e + 1, e)
    t = m - 1.0
    s = t / (t + 2.0)
    s2 = s * s
    p = 2.0 * s * (1.0 + s2 * (1.0 / 3.0 + s2 * (1.0 / 5.0 + s2 * (1.0 / 7.0 + s2 * (1.0 / 9.0)))))
    return e.astype(jnp.float32) * LN2 + p


def _sc_body(scores_h, rel_h, eth_h, g_h, rk_h, out_h,
             scores_v, rel_v, eth_v, probs_v, g_v, rk_v, zt_v, qbuf_v,
             cm_v, tp_v, srel_v, outbuf_v):
    wid = lax.axis_index("s") * 2 + lax.axis_index("c")
    lane = lax.iota(jnp.int32, 16)
    zeros_f = jnp.zeros((16,), jnp.float32)
    zeros_i = jnp.zeros((16,), jnp.int32)
    neg_inf = jnp.full((16,), -jnp.inf, jnp.float32)

    pltpu.sync_copy(scores_h.at[pl.ds(wid * RW * M, RW * M)], scores_v)
    pltpu.sync_copy(rel_h.at[pl.ds(wid * RW * M, RW * M)], rel_v)
    pltpu.sync_copy(eth_h.at[pl.ds(wid * RW * M * G, RW * M * G)], eth_v)

    for bat in range(2):
        rbase = (bat * 16 + lane) * M

        def mx_body(j, mv):
            jv = jnp.full((16,), j, jnp.int32)
            return jnp.maximum(mv, plsc.load_gather(scores_v, [rbase + jv]))
        mv = lax.fori_loop(0, M, mx_body, neg_inf, unroll=8)

        def ex_body(j, carry):
            te, sr = carry
            jv = jnp.full((16,), j, jnp.int32)
            v = plsc.load_gather(scores_v, [rbase + jv])
            ev = jnp.exp(v - mv)
            plsc.store_scatter(probs_v, [rbase + jv], ev)
            rv = plsc.load_gather(rel_v, [rbase + jv])
            return (te + ev, sr + rv)
        te, sr = lax.fori_loop(0, M, ex_body, (zeros_f, zeros_f), unroll=8)

        inv_te = 1.0 / te

        def pscale(j, tp):
            jv = jnp.full((16,), j, jnp.int32)
            pv = plsc.load_gather(probs_v, [rbase + jv]) * inv_te
            plsc.store_scatter(probs_v, [rbase + jv], pv)
            return tp + pv
        tp = lax.fori_loop(0, M, pscale, zeros_f, unroll=8)

        tp_v[pl.ds(bat * 16, 16)] = tp
        srel_v[pl.ds(bat * 16, 16)] = sr

    lbase = lane * M

    def grp_body(grp, acc):
        pair0 = wid * PAIRS + grp * 16
        pltpu.sync_copy(g_h.at[pl.ds(pair0 * M, 16 * M)], g_v)
        pltpu.sync_copy(rk_h.at[pl.ds(pair0 * K, 16 * K)], rk_v)
        p_loc = grp * 16 + lane
        bl = p_loc // NUM_MC
        blM = bl * M

        def zb(c, carry):
            lb0, sb0 = carry
            m = neg_inf
            for t in range(CH):
                gv = plsc.load_gather(g_v, [lb0 + t])
                sv = plsc.load_gather(scores_v, [sb0 + t])
                z = gv + sv
                zt_v[pl.ds(pl.multiple_of(c * (CH * 16) + t * 16, 16), 16)] = z
                m = jnp.maximum(m, z)
            cm_v[pl.ds(pl.multiple_of(c * 16, 16), 16)] = m
            return (lb0 + CH, sb0 + CH)
        lax.fori_loop(0, NCH, zb, (lbase, blM))

        best_idx = []
        q_list = []
        r_list = []
        for i in range(K):
            bcv = neg_inf
            bci = zeros_i
            for c in range(NCH):
                v = cm_v[pl.ds(c * 16, 16)]
                gt = v > bcv
                bcv = jnp.where(gt, v, bcv)
                bci = jnp.where(gt, c, bci)
            zbase = bci * (CH * 16) + lane
            m1 = neg_inf
            t1 = zeros_i
            m2 = neg_inf
            for t in range(CH):
                v = plsc.load_gather(zt_v, [zbase + t * 16])
                gt1 = v > m1
                m2 = jnp.where(gt1, m1, jnp.maximum(m2, v))
                m1 = jnp.where(gt1, v, m1)
                t1 = jnp.where(gt1, t, t1)
            bi = bci * CH + t1
            if i < K - 1:
                plsc.store_scatter(zt_v, [zbase + t1 * 16], neg_inf)
                plsc.store_scatter(cm_v, [bci * 16 + lane], m2)
            best_idx.append(bi)
            qi = plsc.load_gather(probs_v, [blM + bi])
            rk_i = plsc.load_gather(rk_v, [lane * K + i])
            plsc.store_scatter(qbuf_v, [rk_i * 16 + lane], qi)
            q_list.append(qi)
            r_list.append(plsc.load_gather(rel_v, [blM + bi]))

        tp_l = plsc.load_gather(tp_v, [bl])
        srel_l = plsc.load_gather(srel_v, [bl])
        inv_srel = jnp.where(srel_l > 0, 1.0 / srel_l, 0.0)
        T_r = srel_l * inv_srel

        sr_top = r_list[0]
        for i in range(1, K):
            sr_top = sr_top + r_list[i]
        delta = 2.0 * (sr_top * inv_srel) - T_r

        pos = sr_top > 0
        inv_sr = jnp.where(pos, 1.0 / sr_top, 0.0)
        w = [jnp.where(pos, r_list[i] * inv_sr, 1.0 / K) for i in range(K)]

        f_list = []
        for gd in range(G):
            fg = zeros_f
            for i in range(K):
                fg = fg + w[i] * plsc.load_gather(
                    eth_v, [(blM + best_idx[i]) * G + gd])
            f_list.append(fg)
        SF = f_list[0]
        for gd in range(1, G):
            SF = SF + f_list[gd]
        inv_SF = 1.0 / SF
        H = zeros_f
        for gd in range(G):
            pj = f_list[gd] * inv_SF
            H = H - jnp.where(pj > 0, pj * _log16(pj), 0.0)

        logq = zeros_f
        logD = zeros_f
        accq = zeros_f
        for i in range(K):
            qp = qbuf_v[pl.ds(i * 16, 16)]
            logq = logq + _log16(qp)
            logD = logD + _log16(tp_l - accq)
            accq = accq + qp
        logprob = LOG_KFACT + logq - logD

        reward = delta + LAM * H
        return acc + logprob * reward

    acc = lax.fori_loop(0, NGRP, grp_body, zeros_f)
    outbuf_v[...] = acc
    pltpu.sync_copy(outbuf_v, out_h.at[wid])


def kernel(scores, relevance, eth_list):
    key = jax.random.key(1234)
    k_sample, k_perm = jax.random.split(key)
    g = jax.random.gumbel(k_sample, (B, NUM_MC, M), dtype=jnp.float32)
    u = jax.random.uniform(k_perm, (B, NUM_MC, K))
    ua = u[..., :, None]
    ub = u[..., None, :]
    iot = jnp.arange(K)
    lt = ub < ua
    eq = (ub == ua) & (iot[None, :] < iot[:, None])
    ranks = jnp.sum(lt.astype(jnp.int32) + eq.astype(jnp.int32), axis=-1)

    mesh = plsc.VectorSubcoreMesh(core_axis_name="c", subcore_axis_name="s")
    run = functools.partial(
        pl.kernel,
        mesh=mesh,
        compiler_params=pltpu.CompilerParams(needs_layout_passes=False),
        out_type=jax.ShapeDtypeStruct((NW, 16), jnp.float32),
        scratch_types=[
            pltpu.VMEM((RW * M,), jnp.float32),
            pltpu.VMEM((RW * M,), jnp.float32),
            pltpu.VMEM((RW * M * G,), jnp.float32),
            pltpu.VMEM((RW * M,), jnp.float32),
            pltpu.VMEM((16 * M,), jnp.float32),
            pltpu.VMEM((16 * K,), jnp.int32),
            pltpu.VMEM((M * 16,), jnp.float32),
            pltpu.VMEM((K * 16,), jnp.float32),
            pltpu.VMEM((NCH * 16,), jnp.float32),
            pltpu.VMEM((RW,), jnp.float32),
            pltpu.VMEM((RW,), jnp.float32),
            pltpu.VMEM((16,), jnp.float32),
        ],
    )(_sc_body)
    part = run(
        scores.reshape(B * M),
        relevance.reshape(B * M),
        eth_list.reshape(B * M * G),
        g.reshape(B * NUM_MC * M),
        ranks.reshape(B * NUM_MC * K),
    )
    return -jnp.sum(part) / (NUM_MC * B)


# speedup vs baseline: 2.5496x; 1.0595x over previous
import math
import functools

import jax
import jax.numpy as jnp
from jax import lax
from jax.experimental import pallas as pl
from jax.experimental.pallas import tpu as pltpu
from jax.experimental.pallas import tpu_sc as plsc

B = 1024
M = 200
G = 8
K = 10
LAM = 0.5
NUM_MC = 25

NW = 32
RW = B // NW
PAIRS = RW * NUM_MC
NGRP = PAIRS // 16
CH = 20
NCH = M // CH
LOG_KFACT = math.log(float(math.factorial(K)))
LN2 = 0.6931471805599453


def _log16(x):
    b = lax.bitcast_convert_type(x, jnp.int32)
    e = (b >> 23) - 127
    m = lax.bitcast_convert_type((b & 0x007FFFFF) | 0x3F800000, jnp.float32)
    big = m > 1.4142135
    m = jnp.where(big, m * 0.5, m)
    e = jnp.where(big, e + 1, e)
    t = m - 1.0
    s = t / (t + 2.0)
    s2 = s * s
    p = 2.0 * s * (1.0 + s2 * (1.0 / 3.0 + s2 * (1.0 / 5.0 + s2 * (1.0 / 7.0 + s2 * (1.0 / 9.0)))))
    return e.astype(jnp.float32) * LN2 + p


def _sc_body(scores_h, rel_h, eth_h, g_h, u_h, out_h,
             scores_v, rel_v, eth_v, probs_v, g_v, u_v, zt_v, qbuf_v,
             cm_v, tp_v, srel_v, outbuf_v):
    wid = lax.axis_index("s") * 2 + lax.axis_index("c")
    lane = lax.iota(jnp.int32, 16)
    zeros_f = jnp.zeros((16,), jnp.float32)
    zeros_i = jnp.zeros((16,), jnp.int32)
    ones_i = jnp.full((16,), 1, jnp.int32)
    neg_inf = jnp.full((16,), -jnp.inf, jnp.float32)

    pltpu.sync_copy(scores_h.at[pl.ds(wid * RW, RW)], scores_v)
    pltpu.sync_copy(rel_h.at[pl.ds(wid * RW, RW)], rel_v)
    pltpu.sync_copy(eth_h.at[pl.ds(wid * RW, RW)], eth_v)

    for bat in range(2):
        rvec = bat * 16 + lane

        def mx_body(j, mv):
            jv = jnp.full((16,), j, jnp.int32)
            return jnp.maximum(mv, plsc.load_gather(scores_v, [rvec, jv]))
        mv = lax.fori_loop(0, M, mx_body, neg_inf, unroll=8)

        def ex_body(j, carry):
            te, sr = carry
            jv = jnp.full((16,), j, jnp.int32)
            v = plsc.load_gather(scores_v, [rvec, jv])
            ev = jnp.exp(v - mv)
            plsc.store_scatter(probs_v, [rvec, jv], ev)
            rv = plsc.load_gather(rel_v, [rvec, jv])
            return (te + ev, sr + rv)
        te, sr = lax.fori_loop(0, M, ex_body, (zeros_f, zeros_f), unroll=8)

        inv_te = 1.0 / te

        def pscale(j, tp):
            jv = jnp.full((16,), j, jnp.int32)
            pv = plsc.load_gather(probs_v, [rvec, jv]) * inv_te
            plsc.store_scatter(probs_v, [rvec, jv], pv)
            return tp + pv
        tp = lax.fori_loop(0, M, pscale, zeros_f, unroll=8)

        tp_v[pl.ds(bat * 16, 16)] = tp
        srel_v[pl.ds(bat * 16, 16)] = sr

    def grp_body(grp, acc):
        pair0 = wid * PAIRS + grp * 16
        pltpu.sync_copy(g_h.at[pl.ds(pair0, 16)], g_v)
        pltpu.sync_copy(u_h.at[pl.ds(pair0, 16)], u_v)
        p_loc = grp * 16 + lane
        bl = p_loc // NUM_MC

        def zb(c, jv):
            m = neg_inf
            for t in range(CH):
                gv = plsc.load_gather(g_v, [lane, jv + t])
                sv = plsc.load_gather(scores_v, [bl, jv + t])
                z = gv + sv
                zt_v[pl.ds(pl.multiple_of(c * (CH * 16) + t * 16, 16), 16)] = z
                m = jnp.maximum(m, z)
            cm_v[pl.ds(pl.multiple_of(c * 16, 16), 16)] = m
            return jv + CH
        lax.fori_loop(0, NCH, zb, zeros_i)

        u_list = [plsc.load_gather(u_v, [lane, jnp.full((16,), i, jnp.int32)])
                  for i in range(K)]
        rk_list = []
        for i in range(K):
            rk = zeros_i
            for j in range(K):
                if j == i:
                    continue
                lt = u_list[j] < u_list[i]
                if j < i:
                    lt = lt | (u_list[j] == u_list[i])
                rk = rk + jnp.where(lt, ones_i, zeros_i)
            rk_list.append(rk)

        best_idx = []
        r_list = []
        for i in range(K):
            bcv = neg_inf
            bci = zeros_i
            for c in range(NCH):
                v = cm_v[pl.ds(c * 16, 16)]
                gt = v > bcv
                bcv = jnp.where(gt, v, bcv)
                bci = jnp.where(gt, c, bci)
            zbase = bci * (CH * 16) + lane
            m1 = neg_inf
            t1 = zeros_i
            m2 = neg_inf
            for t in range(CH):
                v = plsc.load_gather(zt_v, [zbase + t * 16])
                gt1 = v > m1
                m2 = jnp.where(gt1, m1, jnp.maximum(m2, v))
                m1 = jnp.where(gt1, v, m1)
                t1 = jnp.where(gt1, t, t1)
            bi = bci * CH + t1
            if i < K - 1:
                plsc.store_scatter(zt_v, [zbase + t1 * 16], neg_inf)
                plsc.store_scatter(cm_v, [bci * 16 + lane], m2)
            best_idx.append(bi)
            qi = plsc.load_gather(probs_v, [bl, bi])
            plsc.store_scatter(qbuf_v, [rk_list[i] * 16 + lane], qi)
            r_list.append(plsc.load_gather(rel_v, [bl, bi]))

        tp_l = plsc.load_gather(tp_v, [bl])
        srel_l = plsc.load_gather(srel_v, [bl])
        inv_srel = jnp.where(srel_l > 0, 1.0 / srel_l, 0.0)
        T_r = srel_l * inv_srel

        sr_top = r_list[0]
        for i in range(1, K):
            sr_top = sr_top + r_list[i]
        delta = 2.0 * (sr_top * inv_srel) - T_r

        pos = sr_top > 0
        inv_sr = jnp.where(pos, 1.0 / sr_top, 0.0)
        w = [jnp.where(pos, r_list[i] * inv_sr, 1.0 / K) for i in range(K)]

        f_list = []
        for gd in range(G):
            gv = jnp.full((16,), gd, jnp.int32)
            fg = zeros_f
            for i in range(K):
                fg = fg + w[i] * plsc.load_gather(eth_v, [bl, best_idx[i], gv])
            f_list.append(fg)
        SF = f_list[0]
        for gd in range(1, G):
            SF = SF + f_list[gd]
        inv_SF = 1.0 / SF
        H = zeros_f
        for gd in range(G):
            pj = f_list[gd] * inv_SF
            H = H - jnp.where(pj > 0, pj * _log16(pj), 0.0)

        logq = zeros_f
        logD = zeros_f
        accq = zeros_f
        for i in range(K):
            qp = qbuf_v[pl.ds(i * 16, 16)]
            logq = logq + _log16(qp)
            logD = logD + _log16(tp_l - accq)
            accq = accq + qp
        logprob = LOG_KFACT + logq - logD

        reward = delta + LAM * H
        return acc + logprob * reward

    acc = lax.fori_loop(0, NGRP, grp_body, zeros_f)
    outbuf_v[...] = acc
    pltpu.sync_copy(outbuf_v, out_h.at[wid])


def kernel(scores, relevance, eth_list):
    key = jax.random.key(1234)
    k_sample, k_perm = jax.random.split(key)
    g = jax.random.gumbel(k_sample, (B, NUM_MC, M), dtype=jnp.float32)
    u = jax.random.uniform(k_perm, (B, NUM_MC, K))

    mesh = plsc.VectorSubcoreMesh(core_axis_name="c", subcore_axis_name="s")
    run = functools.partial(
        pl.kernel,
        mesh=mesh,
        compiler_params=pltpu.CompilerParams(
            needs_layout_passes=False, use_tc_tiling_on_sc=False),
        out_type=jax.ShapeDtypeStruct((NW, 16), jnp.float32),
        scratch_types=[
            pltpu.VMEM((RW, M), jnp.float32),
            pltpu.VMEM((RW, M), jnp.float32),
            pltpu.VMEM((RW, M, G), jnp.float32),
            pltpu.VMEM((RW, M), jnp.float32),
            pltpu.VMEM((16, M), jnp.float32),
            pltpu.VMEM((16, K), jnp.float32),
            pltpu.VMEM((M * 16,), jnp.float32),
            pltpu.VMEM((K * 16,), jnp.float32),
            pltpu.VMEM((NCH * 16,), jnp.float32),
            pltpu.VMEM((RW,), jnp.float32),
            pltpu.VMEM((RW,), jnp.float32),
            pltpu.VMEM((16,), jnp.float32),
        ],
    )(_sc_body)
    part = run(
        scores,
        relevance,
        eth_list,
        g.reshape(B * NUM_MC, M),
        u.reshape(B * NUM_MC, K),
    )
    return -jnp.sum(part) / (NUM_MC * B)


# speedup vs baseline: 2.5706x; 1.0083x over previous
import math
import functools

import jax
import jax.numpy as jnp
from jax import lax
from jax.experimental import pallas as pl
from jax.experimental.pallas import tpu as pltpu
from jax.experimental.pallas import tpu_sc as plsc

B = 1024
M = 200
G = 8
K = 10
LAM = 0.5
NUM_MC = 25

NW = 32
RW = B // NW
PAIRS = RW * NUM_MC
NGRP = PAIRS // 16
CH = 20
NCH = M // CH
LOG_KFACT = math.log(float(math.factorial(K)))
LN2 = 0.6931471805599453


def _log16(x):
    b = lax.bitcast_convert_type(x, jnp.int32)
    e = (b >> 23) - 127
    m = lax.bitcast_convert_type((b & 0x007FFFFF) | 0x3F800000, jnp.float32)
    big = m > 1.4142135
    m = jnp.where(big, m * 0.5, m)
    e = jnp.where(big, e + 1, e)
    t = m - 1.0
    s = t / (t + 2.0)
    s2 = s * s
    p = 2.0 * s * (1.0 + s2 * (1.0 / 3.0 + s2 * (1.0 / 5.0 + s2 * (1.0 / 7.0 + s2 * (1.0 / 9.0)))))
    return e.astype(jnp.float32) * LN2 + p


def _sc_body(scores_h, rel_h, eth_h, g_h, u_h, out_h,
             scores_v, rel_v, eth_v, probs_v, g_v, u_v, zt_v, qbuf_v,
             cm_v, tp_v, srel_v, outbuf_v):
    wid = lax.axis_index("s") * 2 + lax.axis_index("c")
    lane = lax.iota(jnp.int32, 16)
    zeros_f = jnp.zeros((16,), jnp.float32)
    zeros_i = jnp.zeros((16,), jnp.int32)
    ones_i = jnp.full((16,), 1, jnp.int32)
    neg_inf = jnp.full((16,), -jnp.inf, jnp.float32)

    pltpu.sync_copy(scores_h.at[pl.ds(wid * RW, RW)], scores_v)
    pltpu.sync_copy(rel_h.at[pl.ds(wid * RW, RW)], rel_v)
    pltpu.sync_copy(eth_h.at[pl.ds(wid * RW, RW)], eth_v)

    for bat in range(2):
        rvec = bat * 16 + lane

        def mx_body(j, mv):
            jv = jnp.full((16,), j, jnp.int32)
            return jnp.maximum(mv, plsc.load_gather(scores_v, [rvec, jv]))
        mv = lax.fori_loop(0, M, mx_body, neg_inf, unroll=8)

        def ex_body(j, carry):
            te, sr = carry
            jv = jnp.full((16,), j, jnp.int32)
            v = plsc.load_gather(scores_v, [rvec, jv])
            ev = jnp.exp(v - mv)
            plsc.store_scatter(probs_v, [rvec, jv], ev)
            rv = plsc.load_gather(rel_v, [rvec, jv])
            return (te + ev, sr + rv)
        te, sr = lax.fori_loop(0, M, ex_body, (zeros_f, zeros_f), unroll=8)

        inv_te = 1.0 / te

        def pscale(j, tp):
            jv = jnp.full((16,), j, jnp.int32)
            pv = plsc.load_gather(probs_v, [rvec, jv]) * inv_te
            plsc.store_scatter(probs_v, [rvec, jv], pv)
            return tp + pv
        tp = lax.fori_loop(0, M, pscale, zeros_f, unroll=8)

        tp_v[pl.ds(bat * 16, 16)] = tp
        srel_v[pl.ds(bat * 16, 16)] = sr

    def grp_body(grp, acc):
        pair0 = wid * PAIRS + grp * 16
        pltpu.sync_copy(g_h.at[pl.ds(pair0, 16)], g_v)
        pltpu.sync_copy(u_h.at[pl.ds(pair0, 16)], u_v)
        p_loc = grp * 16 + lane
        bl = p_loc // NUM_MC

        def zb(c, jv):
            m = neg_inf
            for t in range(CH):
                gv = plsc.load_gather(g_v, [lane, jv + t])
                sv = plsc.load_gather(scores_v, [bl, jv + t])
                z = gv + sv
                zt_v[pl.ds(pl.multiple_of(c * (CH * 16) + t * 16, 16), 16)] = z
                m = jnp.maximum(m, z)
            cm_v[pl.ds(pl.multiple_of(c * 16, 16), 16)] = m
            return jv + CH
        lax.fori_loop(0, NCH, zb, zeros_i)

        u_list = [plsc.load_gather(u_v, [lane, jnp.full((16,), i, jnp.int32)])
                  for i in range(K)]
        rk_list = []
        for i in range(K):
            rk = zeros_i
            for j in range(K):
                if j == i:
                    continue
                lt = u_list[j] < u_list[i]
                if j < i:
                    lt = lt | (u_list[j] == u_list[i])
                rk = rk + jnp.where(lt, ones_i, zeros_i)
            rk_list.append(rk)

        best_idx = []
        r_list = []
        for i in range(K):
            bcv = neg_inf
            bci = zeros_i
            for c in range(NCH):
                v = cm_v[pl.ds(c * 16, 16)]
                gt = v > bcv
                bcv = jnp.where(gt, v, bcv)
                bci = jnp.where(gt, c, bci)
            zbase = bci * (CH * 16) + lane
            m1 = neg_inf
            t1 = zeros_i
            m2 = neg_inf
            for t in range(CH):
                v = plsc.load_gather(zt_v, [zbase + t * 16])
                gt1 = v > m1
                m2 = jnp.where(gt1, m1, jnp.maximum(m2, v))
                m1 = jnp.where(gt1, v, m1)
                t1 = jnp.where(gt1, t, t1)
            bi = bci * CH + t1
            if i < K - 1:
                plsc.store_scatter(zt_v, [zbase + t1 * 16], neg_inf)
                plsc.store_scatter(cm_v, [bci * 16 + lane], m2)
            best_idx.append(bi)
            qi = plsc.load_gather(probs_v, [bl, bi])
            plsc.store_scatter(qbuf_v, [rk_list[i] * 16 + lane], qi)
            r_list.append(plsc.load_gather(rel_v, [bl, bi]))

        tp_l = plsc.load_gather(tp_v, [bl])
        srel_l = plsc.load_gather(srel_v, [bl])
        inv_srel = jnp.where(srel_l > 0, 1.0 / srel_l, 0.0)
        T_r = srel_l * inv_srel

        sr_top = r_list[0]
        for i in range(1, K):
            sr_top = sr_top + r_list[i]
        delta = 2.0 * (sr_top * inv_srel) - T_r

        pos = sr_top > 0
        inv_sr = jnp.where(pos, 1.0 / sr_top, 0.0)
        w = [jnp.where(pos, r_list[i] * inv_sr, 1.0 / K) for i in range(K)]

        f_list = []
        for gd in range(G):
            gv = jnp.full((16,), gd, jnp.int32)
            fg = zeros_f
            for i in range(K):
                fg = fg + w[i] * plsc.load_gather(eth_v, [bl, best_idx[i], gv])
            f_list.append(fg)
        SF = f_list[0]
        for gd in range(1, G):
            SF = SF + f_list[gd]
        inv_SF = 1.0 / SF
        H = zeros_f
        for gd in range(G):
            pj = f_list[gd] * inv_SF
            H = H - jnp.where(pj > 0, pj * _log16(pj), 0.0)

        logq = zeros_f
        logD = zeros_f
        accq = zeros_f
        for i in range(K):
            qp = qbuf_v[pl.ds(i * 16, 16)]
            logq = logq + _log16(qp)
            logD = logD + _log16(tp_l - accq)
            accq = accq + qp
        logprob = LOG_KFACT + logq - logD

        reward = delta + LAM * H
        return acc + logprob * reward

    acc = lax.fori_loop(0, NGRP, grp_body, zeros_f)
    outbuf_v[...] = acc
    pltpu.sync_copy(outbuf_v, out_h.at[wid])


def kernel(scores, relevance, eth_list):
    key = jax.random.key(1234)
    k_sample, k_perm = jax.random.split(key)
    g = jax.random.gumbel(k_sample, (B * NUM_MC, M), dtype=jnp.float32)
    u = jax.random.uniform(k_perm, (B * NUM_MC, K))

    mesh = plsc.VectorSubcoreMesh(core_axis_name="c", subcore_axis_name="s")
    run = functools.partial(
        pl.kernel,
        mesh=mesh,
        compiler_params=pltpu.CompilerParams(
            needs_layout_passes=False, use_tc_tiling_on_sc=False),
        out_type=jax.ShapeDtypeStruct((NW, 16), jnp.float32),
        scratch_types=[
            pltpu.VMEM((RW, M), jnp.float32),
            pltpu.VMEM((RW, M), jnp.float32),
            pltpu.VMEM((RW, M, G), jnp.float32),
            pltpu.VMEM((RW, M), jnp.float32),
            pltpu.VMEM((16, M), jnp.float32),
            pltpu.VMEM((16, K), jnp.float32),
            pltpu.VMEM((M * 16,), jnp.float32),
            pltpu.VMEM((K * 16,), jnp.float32),
            pltpu.VMEM((NCH * 16,), jnp.float32),
            pltpu.VMEM((RW,), jnp.float32),
            pltpu.VMEM((RW,), jnp.float32),
            pltpu.VMEM((16,), jnp.float32),
        ],
    )(_sc_body)
    part = run(scores, relevance, eth_list, g, u)
    return -jnp.sum(part) / (NUM_MC * B)
